# baseline (speedup 1.0000x reference)
import functools

import jax
import jax.numpy as jnp
from jax import lax
from jax.experimental import pallas as pl
from jax.experimental.pallas import tpu as pltpu
from jax.experimental.pallas import tpu_sc as plsc

N = 10000
E = 160000
D_IN = 256
D_HID = 16
D_OUT = 64

L = 16
NC = 2
NS = 16
NW = NC * NS
EPT = 5120
EPAD = NW * EPT
CHUNK = 128
NCH = EPT // CHUNK
NP = 10240
RPT = NP // NS
SP = 128
HEPT = EPT // 2
HCH = NCH // 2
QEPT = EPT // 4
QCH = NCH // 4

R_TC = 512


def _mesh():
    return plsc.VectorSubcoreMesh(core_axis_name="c", subcore_axis_name="s")


def _sc_params():
    return pltpu.CompilerParams(needs_layout_passes=False,
                                use_tc_tiling_on_sc=False)


def _rsqrt_nt(x):
    i = plsc.bitcast(x, jnp.int32)
    i = jnp.int32(0x5F3759DF) - (i >> 1)
    y = plsc.bitcast(i, jnp.float32)
    for _ in range(3):
        y = y * (1.5 - 0.5 * x * y * y)
    return y


def _l1_body(h_hbm, row_hbm, col2_hbm, ew2_hbm, z_hbm,
             part_out, norm_out, dinv_out,
             row_f, cold, ewd, norm_v, dinv_v, deg_v, slab_v, dinv_sl,
             rows_a, rows_b, gsem, ssa, ssb,
             deg_sp, dinv_spm, out_sp):
    c = lax.axis_index("c")
    s = lax.axis_index("s")
    wid = s * NC + c
    eoff = pl.multiple_of(wid * EPT, 128)
    doff = pl.multiple_of(s * 2 * NCH, 8)
    soff = pl.multiple_of(s * RPT, 128)

    pltpu.sync_copy(row_hbm.at[pl.ds(eoff, EPT)], row_f)
    pltpu.sync_copy(col2_hbm.at[pl.ds(doff, 2 * NCH)], cold)
    pltpu.sync_copy(ew2_hbm.at[pl.ds(doff, 2 * NCH)], ewd)

    bufs = (rows_a, rows_b)
    sems = (ssa, ssb)

    def fire_gather(q, buf):
        qoff = pl.multiple_of(q * QEPT, 128)
        pltpu.async_copy(h_hbm.at[row_f.at[pl.ds(qoff, QEPT)]], buf, gsem)

    fire_gather(0, rows_a)

    zero = jnp.zeros((L,), jnp.float32)

    def zbody(i, carry):
        off = pl.multiple_of(i * L, L)
        deg_v[pl.ds(off, L)] = zero
        return carry

    lax.fori_loop(0, NP // L, zbody, 0)
    pltpu.sync_copy(z_hbm.at[pl.ds(soff, RPT)], out_sp.at[pl.ds(soff, RPT)])

    def deg_body(j, carry):
        for k in range(CHUNK // L):
            sl = pl.ds(k * L, L)
            plsc.addupdate_scatter(deg_v, [cold[j, sl]], ewd[j, sl])
        return carry

    lax.fori_loop(0, 2 * NCH, deg_body, 0)
    pltpu.sync_copy(deg_v, deg_sp.at[s])
    plsc.subcore_barrier()

    for t in range(RPT // 64):
        boff = pl.multiple_of(s * RPT + t * 64, 64)
        pltpu.sync_copy(deg_sp.at[:, pl.ds(boff, 64)], slab_v)
        for v in range(4):
            sl = pl.ds(v * L, L)
            acc = slab_v[0, sl]
            for k in range(1, NS):
                acc = acc + slab_v[k, sl]
            dinv_sl[pl.ds(pl.multiple_of(t * 64 + v * L, L), L)] = (
                _rsqrt_nt(acc + 1.0))

    pltpu.sync_copy(dinv_sl, dinv_spm.at[pl.ds(soff, RPT)])
    ooff = pl.multiple_of(c * NP + s * RPT, 128)
    pltpu.sync_copy(dinv_sl, dinv_out.at[pl.ds(ooff, RPT)])
    plsc.subcore_barrier()
    pltpu.sync_copy(dinv_spm.at[pl.ds(0, N)], dinv_v)

    def norm_chunk(j, carry):
        for k in range(CHUNK // L):
            sl = pl.ds(k * L, L)
            noff = pl.multiple_of(j * CHUNK + k * L, L)
            norm_v[pl.ds(noff, L)] = (
                plsc.load_gather(dinv_v, [row_f[pl.ds(noff, L)]])
                * ewd[c * NCH + j, sl]
                * plsc.load_gather(dinv_v, [cold[c * NCH + j, sl]]))
        return carry

    lax.fori_loop(0, NCH, norm_chunk, 0)
    pltpu.sync_copy(norm_v, norm_out.at[pl.ds(eoff, EPT)])

    def do_quarter(q, buf, ssem_q):
        def msg_chunk(j, carry):
            def scale(g, carry2):
                m0 = j * CHUNK + g * L
                noff = pl.multiple_of(q * QEPT + m0, L)
                nv = norm_v[pl.ds(noff, L)]
                for k in range(L):
                    nb = jnp.full((L,), nv[k], jnp.float32)
                    buf[m0 + k] = buf[m0 + k] * nb
                return carry2

            lax.fori_loop(0, CHUNK // L, scale, 0)
            roff = pl.multiple_of(j * CHUNK, CHUNK)
            pltpu.async_copy(buf.at[pl.ds(roff, CHUNK)],
                             out_sp.at[cold.at[c * NCH + q * QCH + j]],
                             ssem_q, add=True)
            return carry

        lax.fori_loop(0, QCH, msg_chunk, 0)

    for q in range(4):
        x = bufs[q % 2]
        pltpu.make_async_copy(h_hbm.at[pl.ds(0, QEPT)], x, gsem).wait()
        if q >= 1:
            y = bufs[(q - 1) % 2]
            pltpu.make_async_copy(
                y, out_sp.at[pl.ds(0, QEPT)], sems[(q - 1) % 2]).wait()
        if q + 1 < 4:
            fire_gather(q + 1, bufs[(q + 1) % 2])
        do_quarter(q, x, sems[q % 2])

    pltpu.make_async_copy(bufs[1], out_sp.at[pl.ds(0, QEPT)], sems[1]).wait()
    plsc.subcore_barrier()
    poff = pl.multiple_of(c * NP + s * RPT, 128)
    pltpu.sync_copy(out_sp.at[pl.ds(soff, RPT)],
                    part_out.at[pl.ds(poff, RPT)])


def _layer1(h_np, row_p, col_p, ew_p, zeros16):
    return pl.kernel(
        _l1_body,
        out_type=(jax.ShapeDtypeStruct((NC * NP, D_HID), jnp.float32),
                  jax.ShapeDtypeStruct((EPAD,), jnp.float32),
                  jax.ShapeDtypeStruct((NC * NP,), jnp.float32)),
        mesh=_mesh(),
        scratch_types=[
            pltpu.VMEM((EPT,), jnp.int32),
            pltpu.VMEM((2 * NCH, CHUNK), jnp.int32),
            pltpu.VMEM((2 * NCH, CHUNK), jnp.float32),
            pltpu.VMEM((EPT,), jnp.float32),
            pltpu.VMEM((N,), jnp.float32),
            pltpu.VMEM((NP,), jnp.float32),
            pltpu.VMEM((NS, 64), jnp.float32),
            pltpu.VMEM((RPT,), jnp.float32),
            pltpu.VMEM((QEPT, D_HID), jnp.float32),
            pltpu.VMEM((QEPT, D_HID), jnp.float32),
            pltpu.SemaphoreType.DMA,
            pltpu.SemaphoreType.DMA,
            pltpu.SemaphoreType.DMA,
            pltpu.VMEM_SHARED((NS, NP), jnp.float32),
            pltpu.VMEM_SHARED((NP,), jnp.float32),
            pltpu.VMEM_SHARED((NP, D_HID), jnp.float32),
        ],
        compiler_params=_sc_params(),
    )(h_np, row_p, col_p, ew_p, zeros16)


def _msg2_body(p1_hbm, h_hbm, dinv_hbm, b1_hbm, norm_hbm, row_hbm, col_hbm,
               z_hbm, part_out,
               row_v, col_v, norm_v, rows_v, p0s, p1s, hs, dinv_sl, b1_v,
               gsem, ssem, h1_sp, out_sp):
    c = lax.axis_index("c")
    s = lax.axis_index("s")
    wid = s * NC + c
    ecoff = pl.multiple_of(wid * NCH, 8)
    eoff = pl.multiple_of(wid * EPT, 128)

    pltpu.sync_copy(row_hbm.at[pl.ds(eoff, EPT)], row_v)
    pltpu.sync_copy(col_hbm.at[pl.ds(ecoff, NCH)], col_v)
    pltpu.sync_copy(norm_hbm.at[pl.ds(eoff, EPT)], norm_v)

    soff = pl.multiple_of(s * RPT, 128)
    pltpu.sync_copy(b1_hbm, b1_v)
    b1r = b1_v[...]

    for t in range(RPT // SP):
        rbase = pl.multiple_of(s * RPT + t * SP, 128)
        pltpu.sync_copy(p1_hbm.at[pl.ds(rbase, SP)], p0s)
        pltpu.sync_copy(
            p1_hbm.at[pl.ds(pl.multiple_of(NP + s * RPT + t * SP, 128), SP)],
            p1s)
        pltpu.sync_copy(h_hbm.at[pl.ds(rbase, SP)], hs)
        pltpu.sync_copy(dinv_hbm.at[pl.ds(rbase, SP)], dinv_sl)

        def h1_body(g, carry):
            goff = pl.multiple_of(g * L, L)
            dv = dinv_sl[pl.ds(goff, L)]
            dsq = dv * dv
            for k in range(L):
                r = g * L + k
                h1r = (p0s[r] + p1s[r]
                       + jnp.full((L,), dsq[k], jnp.float32) * hs[r] + b1r)
                hs[r] = jnp.maximum(h1r, 0.0)
            return carry

        lax.fori_loop(0, SP // L, h1_body, 0)
        pltpu.sync_copy(hs, h1_sp.at[pl.ds(rbase, SP)])

    pltpu.sync_copy(z_hbm.at[pl.ds(soff, RPT)], out_sp.at[pl.ds(soff, RPT)])
    plsc.subcore_barrier()

    for hh in range(2):
        hoff = pl.multiple_of(hh * HEPT, 128)
        pltpu.async_copy(h1_sp.at[row_v.at[pl.ds(hoff, HEPT)]], rows_v, gsem)
        pltpu.make_async_copy(p1_hbm.at[pl.ds(0, HEPT)], rows_v, gsem).wait()

        def msg_chunk(j, carry):
            def scale(g, carry2):
                m0 = j * CHUNK + g * L
                noff = pl.multiple_of(hh * HEPT + m0, L)
                nv = norm_v[pl.ds(noff, L)]
                for k in range(L):
                    nb = jnp.full((L,), nv[k], jnp.float32)
                    rows_v[m0 + k] = rows_v[m0 + k] * nb
                return carry2

            lax.fori_loop(0, CHUNK // L, scale, 0)
            roff = pl.multiple_of(j * CHUNK, CHUNK)
            pltpu.async_copy(rows_v.at[pl.ds(roff, CHUNK)],
                             out_sp.at[col_v.at[hh * HCH + j]], ssem,
                             add=True)
            return carry

        lax.fori_loop(0, HCH, msg_chunk, 0)
        pltpu.make_async_copy(rows_v, out_sp.at[pl.ds(0, HEPT)], ssem).wait()

    plsc.subcore_barrier()
    poff = pl.multiple_of(c * NP + s * RPT, 128)
    pltpu.sync_copy(out_sp.at[pl.ds(soff, RPT)],
                    part_out.at[pl.ds(poff, RPT)])


def _msg_layer2(p1, h_np, dinv_full, b1, norm, row_p, col_p, zeros16):
    return pl.kernel(
        _msg2_body,
        out_type=jax.ShapeDtypeStruct((NC * NP, D_HID), jnp.float32),
        mesh=_mesh(),
        scratch_types=[
            pltpu.VMEM((EPT,), jnp.int32),
            pltpu.VMEM((NCH, CHUNK), jnp.int32),
            pltpu.VMEM((EPT,), jnp.float32),
            pltpu.VMEM((HEPT, D_HID), jnp.float32),
            pltpu.VMEM((SP, D_HID), jnp.float32),
            pltpu.VMEM((SP, D_HID), jnp.float32),
            pltpu.VMEM((SP, D_HID), jnp.float32),
            pltpu.VMEM((SP,), jnp.float32),
            pltpu.VMEM((L,), jnp.float32),
            pltpu.SemaphoreType.DMA,
            pltpu.SemaphoreType.DMA,
            pltpu.VMEM_SHARED((NP, D_HID), jnp.float32),
            pltpu.VMEM_SHARED((NP, D_HID), jnp.float32),
        ],
        compiler_params=_sc_params(),
    )(p1, h_np, dinv_full, b1, norm, row_p, col_p, zeros16)


def _tc1_body(x_ref, w1_ref, h_ref):
    h_ref[...] = jnp.dot(x_ref[...], w1_ref[...],
                         preferred_element_type=jnp.float32)


def _tc1(x, W1):
    grid = NP // R_TC
    return pl.pallas_call(
        _tc1_body,
        grid=(grid,),
        in_specs=[
            pl.BlockSpec((R_TC, D_IN), lambda i: (i, 0)),
            pl.BlockSpec((D_IN, D_HID), lambda i: (0, 0)),
        ],
        out_specs=pl.BlockSpec((R_TC, D_HID), lambda i: (i, 0)),
        out_shape=jax.ShapeDtypeStruct((NP, D_HID), jnp.float32),
    )(x, W1)


def _tc3_body(p1_ref, p2_ref, h_ref, dinv_ref, b1_ref, w2_ref, b2_ref,
              out_ref):
    dsq = dinv_ref[...] * dinv_ref[...]
    h1 = jnp.maximum(
        p1_ref[0] + p1_ref[1] + dsq * h_ref[...] + b1_ref[...], 0.0)
    agg = p2_ref[0] + p2_ref[1] + dsq * h1
    z = jnp.dot(agg, w2_ref[...], preferred_element_type=jnp.float32)
    z = z + b2_ref[...]
    m = jnp.max(z, axis=1, keepdims=True)
    zs = z - m
    lse = jnp.log(jnp.sum(jnp.exp(zs), axis=1, keepdims=True))
    out_ref[...] = zs - lse


def _tc3(p1, p2, h_np, dinv_col, b1_row, W2, b2_row):
    grid = (N + R_TC - 1) // R_TC
    return pl.pallas_call(
        _tc3_body,
        grid=(grid,),
        in_specs=[
            pl.BlockSpec((NC, R_TC, D_HID), lambda i: (0, i, 0)),
            pl.BlockSpec((NC, R_TC, D_HID), lambda i: (0, i, 0)),
            pl.BlockSpec((R_TC, D_HID), lambda i: (i, 0)),
            pl.BlockSpec((R_TC, 1), lambda i: (i, 0)),
            pl.BlockSpec((1, D_HID), lambda i: (0, 0)),
            pl.BlockSpec((D_HID, D_OUT), lambda i: (0, 0)),
            pl.BlockSpec((1, D_OUT), lambda i: (0, 0)),
        ],
        out_specs=pl.BlockSpec((R_TC, D_OUT), lambda i: (i, 0)),
        out_shape=jax.ShapeDtypeStruct((N, D_OUT), jnp.float32),
    )(p1, p2, h_np, dinv_col, b1_row, W2, b2_row)


def kernel(x, edge_index, edge_weight, W1, b1, W2, b2):
    row = edge_index[0].astype(jnp.int32)
    col = edge_index[1].astype(jnp.int32)
    ew = edge_weight.astype(jnp.float32)

    pad = EPAD - E
    row_p = jnp.concatenate([row, jnp.zeros((pad,), jnp.int32)])
    col_p = jnp.concatenate([col, jnp.zeros((pad,), jnp.int32)])
    ew_p = jnp.concatenate([ew, jnp.zeros((pad,), jnp.float32)])
    col_p = col_p.reshape(NW * NCH, CHUNK)
    ew_p = ew_p.reshape(NW * NCH, CHUNK)

    h = _tc1(x, W1)
    zeros16 = jnp.zeros((NP, D_HID), jnp.float32)
    p1, norm, dinv_full = _layer1(h, row_p, col_p, ew_p, zeros16)
    dinv_col = dinv_full[:N].reshape(N, 1)
    p2 = _msg_layer2(p1, h, dinv_full, b1, norm, row_p, col_p, zeros16)
    return _tc3(p1.reshape(NC, NP, D_HID), p2.reshape(NC, NP, D_HID),
                h, dinv_col, b1.reshape(1, D_HID), W2,
                b2.reshape(1, D_OUT))


# speedup vs baseline: 1.0642x; 1.0642x over previous
import functools

import jax
import jax.numpy as jnp
from jax import lax
from jax.experimental import pallas as pl
from jax.experimental.pallas import tpu as pltpu
from jax.experimental.pallas import tpu_sc as plsc

N = 10000
E = 160000
D_IN = 256
D_HID = 16
D_OUT = 64

L = 16
NC = 2
NS = 16
NW = NC * NS
EPT = 5120
EPAD = NW * EPT
CHUNK = 128
NCH = EPT // CHUNK
NP = 10240
RPT = NP // NS
SP = 128
HEPT = EPT // 2
HCH = NCH // 2
QEPT = EPT // 4
QCH = NCH // 4

R_TC = 512


def _mesh():
    return plsc.VectorSubcoreMesh(core_axis_name="c", subcore_axis_name="s")


def _sc_params():
    return pltpu.CompilerParams(needs_layout_passes=False,
                                use_tc_tiling_on_sc=False)


def _quartered_msg(norm_v, cold, col_base, out_sp, bufs, sems, gsem,
                   fire_gather, drain_src):
    def do_quarter(q, buf, ssem_q):
        def msg_chunk(j, carry):
            def scale(g, carry2):
                m0 = j * CHUNK + g * L
                noff = pl.multiple_of(q * QEPT + m0, L)
                nv = norm_v[pl.ds(noff, L)]
                for k in range(L):
                    nb = jnp.full((L,), nv[k], jnp.float32)
                    buf[m0 + k] = buf[m0 + k] * nb
                return carry2

            lax.fori_loop(0, CHUNK // L, scale, 0)
            roff = pl.multiple_of(j * CHUNK, CHUNK)
            pltpu.async_copy(buf.at[pl.ds(roff, CHUNK)],
                             out_sp.at[cold.at[col_base + q * QCH + j]],
                             ssem_q, add=True)
            return carry

        lax.fori_loop(0, QCH, msg_chunk, 0)

    for q in range(4):
        x = bufs[q % 2]
        pltpu.make_async_copy(drain_src, x, gsem).wait()
        if q >= 1:
            y = bufs[(q - 1) % 2]
            pltpu.make_async_copy(
                y, out_sp.at[pl.ds(0, QEPT)], sems[(q - 1) % 2]).wait()
        if q + 1 < 4:
            fire_gather(q + 1, bufs[(q + 1) % 2])
        do_quarter(q, x, sems[q % 2])

    pltpu.make_async_copy(bufs[1], out_sp.at[pl.ds(0, QEPT)], sems[1]).wait()


def _rsqrt_nt(x):
    i = plsc.bitcast(x, jnp.int32)
    i = jnp.int32(0x5F3759DF) - (i >> 1)
    y = plsc.bitcast(i, jnp.float32)
    for _ in range(3):
        y = y * (1.5 - 0.5 * x * y * y)
    return y


def _dinv_body(col_hbm, ew_hbm, dinv_out,
               col_v, ew_v, deg_v, slab_v, dinv_sl, deg_sp):
    c = lax.axis_index("c")
    s = lax.axis_index("s")
    doff = pl.multiple_of(s * 2 * NCH, 8)
    pltpu.sync_copy(col_hbm.at[pl.ds(doff, 2 * NCH)], col_v)
    pltpu.sync_copy(ew_hbm.at[pl.ds(doff, 2 * NCH)], ew_v)

    zero = jnp.zeros((L,), jnp.float32)

    def zbody(i, carry):
        off = pl.multiple_of(i * L, L)
        deg_v[pl.ds(off, L)] = zero
        return carry

    lax.fori_loop(0, NP // L, zbody, 0)

    def chunk_body(j, carry):
        for k in range(CHUNK // L):
            sl = pl.ds(k * L, L)
            plsc.addupdate_scatter(deg_v, [col_v[j, sl]], ew_v[j, sl])
        return carry

    lax.fori_loop(0, 2 * NCH, chunk_body, 0)
    pltpu.sync_copy(deg_v, deg_sp.at[s])
    plsc.subcore_barrier()

    coff = pl.multiple_of(s * RPT, 128)
    pltpu.sync_copy(deg_sp.at[:, pl.ds(coff, RPT)], slab_v)

    def red_body(v, carry):
        voff = pl.multiple_of(v * L, L)
        acc = slab_v[0, pl.ds(voff, L)]
        for k in range(1, NS):
            acc = acc + slab_v[k, pl.ds(voff, L)]
        dinv_sl[pl.ds(voff, L)] = _rsqrt_nt(acc + 1.0)
        return carry

    lax.fori_loop(0, RPT // L, red_body, 0)
    ooff = pl.multiple_of(c * NP + s * RPT, 128)
    pltpu.sync_copy(dinv_sl, dinv_out.at[pl.ds(ooff, RPT)])


def _dinv_kernel(col_p, ew_p):
    return pl.kernel(
        _dinv_body,
        out_type=jax.ShapeDtypeStruct((NC * NP,), jnp.float32),
        mesh=_mesh(),
        scratch_types=[
            pltpu.VMEM((2 * NCH, CHUNK), jnp.int32),
            pltpu.VMEM((2 * NCH, CHUNK), jnp.float32),
            pltpu.VMEM((NP,), jnp.float32),
            pltpu.VMEM((NS, RPT), jnp.float32),
            pltpu.VMEM((RPT,), jnp.float32),
            pltpu.VMEM_SHARED((NS, NP), jnp.float32),
        ],
        compiler_params=_sc_params(),
    )(col_p, ew_p)


def _msgn_body(h_hbm, dinv_hbm, row_hbm, col_hbm, ew_hbm, z_hbm,
               part_out, norm_out,
               row_f, cold, ewd, norm_v, dinv_v, rows_a, rows_b,
               gsem, ssa, ssb, out_sp):
    c = lax.axis_index("c")
    s = lax.axis_index("s")
    wid = s * NC + c
    eoff = pl.multiple_of(wid * EPT, 128)
    ecoff = pl.multiple_of(wid * NCH, 8)
    soff = pl.multiple_of(s * RPT, 128)

    pltpu.sync_copy(row_hbm.at[pl.ds(eoff, EPT)], row_f)
    pltpu.sync_copy(col_hbm.at[pl.ds(ecoff, NCH)], cold)
    pltpu.sync_copy(ew_hbm.at[pl.ds(ecoff, NCH)], ewd)
    pltpu.sync_copy(dinv_hbm, dinv_v)

    bufs = (rows_a, rows_b)
    sems = (ssa, ssb)

    def fire_gather(q, buf):
        qoff = pl.multiple_of(q * QEPT, 128)
        pltpu.async_copy(h_hbm.at[row_f.at[pl.ds(qoff, QEPT)]], buf, gsem)

    fire_gather(0, rows_a)
    pltpu.sync_copy(z_hbm.at[pl.ds(soff, RPT)], out_sp.at[pl.ds(soff, RPT)])

    def norm_chunk(j, carry):
        for k in range(CHUNK // L):
            sl = pl.ds(k * L, L)
            noff = pl.multiple_of(j * CHUNK + k * L, L)
            norm_v[pl.ds(noff, L)] = (
                plsc.load_gather(dinv_v, [row_f[pl.ds(noff, L)]])
                * ewd[j, sl]
                * plsc.load_gather(dinv_v, [cold[j, sl]]))
        return carry

    lax.fori_loop(0, NCH, norm_chunk, 0)
    pltpu.sync_copy(norm_v, norm_out.at[pl.ds(eoff, EPT)])

    plsc.subcore_barrier()
    _quartered_msg(norm_v, cold, 0, out_sp, bufs, sems, gsem,
                   fire_gather, h_hbm.at[pl.ds(0, QEPT)])
    plsc.subcore_barrier()
    poff = pl.multiple_of(c * NP + s * RPT, 128)
    pltpu.sync_copy(out_sp.at[pl.ds(soff, RPT)],
                    part_out.at[pl.ds(poff, RPT)])


def _msgn(h_np, dinv_flat, row_p, col_p, ew_p, zeros16):
    return pl.kernel(
        _msgn_body,
        out_type=(jax.ShapeDtypeStruct((NC * NP, D_HID), jnp.float32),
                  jax.ShapeDtypeStruct((EPAD,), jnp.float32)),
        mesh=_mesh(),
        scratch_types=[
            pltpu.VMEM((EPT,), jnp.int32),
            pltpu.VMEM((NCH, CHUNK), jnp.int32),
            pltpu.VMEM((NCH, CHUNK), jnp.float32),
            pltpu.VMEM((EPT,), jnp.float32),
            pltpu.VMEM((N,), jnp.float32),
            pltpu.VMEM((QEPT, D_HID), jnp.float32),
            pltpu.VMEM((QEPT, D_HID), jnp.float32),
            pltpu.SemaphoreType.DMA,
            pltpu.SemaphoreType.DMA,
            pltpu.SemaphoreType.DMA,
            pltpu.VMEM_SHARED((NP, D_HID), jnp.float32),
        ],
        compiler_params=_sc_params(),
    )(h_np, dinv_flat, row_p, col_p, ew_p, zeros16)


def _msg2_body(p1_hbm, h_hbm, dinv_hbm, b1_hbm, norm_hbm, row_hbm, col_hbm,
               z_hbm, part_out,
               row_v, col_v, norm_v, rows_a, rows_b, p0s, p1s, hs, dinv_sl,
               b1_v, gsem, ssa, ssb, h1_sp, out_sp):
    c = lax.axis_index("c")
    s = lax.axis_index("s")
    wid = s * NC + c
    ecoff = pl.multiple_of(wid * NCH, 8)
    eoff = pl.multiple_of(wid * EPT, 128)

    pltpu.sync_copy(row_hbm.at[pl.ds(eoff, EPT)], row_v)
    pltpu.sync_copy(col_hbm.at[pl.ds(ecoff, NCH)], col_v)
    pltpu.sync_copy(norm_hbm.at[pl.ds(eoff, EPT)], norm_v)

    soff = pl.multiple_of(s * RPT, 128)
    pltpu.sync_copy(b1_hbm, b1_v)
    b1r = b1_v[...]

    for t in range(RPT // SP):
        rbase = pl.multiple_of(s * RPT + t * SP, 128)
        pltpu.sync_copy(p1_hbm.at[pl.ds(rbase, SP)], p0s)
        pltpu.sync_copy(
            p1_hbm.at[pl.ds(pl.multiple_of(NP + s * RPT + t * SP, 128), SP)],
            p1s)
        pltpu.sync_copy(h_hbm.at[pl.ds(rbase, SP)], hs)
        pltpu.sync_copy(dinv_hbm.at[pl.ds(rbase, SP)], dinv_sl)

        def h1_body(g, carry):
            goff = pl.multiple_of(g * L, L)
            dv = dinv_sl[pl.ds(goff, L)]
            dsq = dv * dv
            for k in range(L):
                r = g * L + k
                h1r = (p0s[r] + p1s[r]
                       + jnp.full((L,), dsq[k], jnp.float32) * hs[r] + b1r)
                hs[r] = jnp.maximum(h1r, 0.0)
            return carry

        lax.fori_loop(0, SP // L, h1_body, 0)
        pltpu.sync_copy(hs, h1_sp.at[pl.ds(rbase, SP)])

    pltpu.sync_copy(z_hbm.at[pl.ds(soff, RPT)], out_sp.at[pl.ds(soff, RPT)])
    plsc.subcore_barrier()

    bufs = (rows_a, rows_b)
    sems = (ssa, ssb)

    def fire_gather(q, buf):
        qoff = pl.multiple_of(q * QEPT, 128)
        pltpu.async_copy(h1_sp.at[row_v.at[pl.ds(qoff, QEPT)]], buf, gsem)

    fire_gather(0, rows_a)
    _quartered_msg(norm_v, col_v, 0, out_sp, bufs, sems, gsem,
                   fire_gather, p1_hbm.at[pl.ds(0, QEPT)])
    plsc.subcore_barrier()
    poff = pl.multiple_of(c * NP + s * RPT, 128)
    pltpu.sync_copy(out_sp.at[pl.ds(soff, RPT)],
                    part_out.at[pl.ds(poff, RPT)])


def _msg_layer2(p1, h_np, dinv_full, b1, norm, row_p, col_p, zeros16):
    return pl.kernel(
        _msg2_body,
        out_type=jax.ShapeDtypeStruct((NC * NP, D_HID), jnp.float32),
        mesh=_mesh(),
        scratch_types=[
            pltpu.VMEM((EPT,), jnp.int32),
            pltpu.VMEM((NCH, CHUNK), jnp.int32),
            pltpu.VMEM((EPT,), jnp.float32),
            pltpu.VMEM((QEPT, D_HID), jnp.float32),
            pltpu.VMEM((QEPT, D_HID), jnp.float32),
            pltpu.VMEM((SP, D_HID), jnp.float32),
            pltpu.VMEM((SP, D_HID), jnp.float32),
            pltpu.VMEM((SP, D_HID), jnp.float32),
            pltpu.VMEM((SP,), jnp.float32),
            pltpu.VMEM((L,), jnp.float32),
            pltpu.SemaphoreType.DMA,
            pltpu.SemaphoreType.DMA,
            pltpu.SemaphoreType.DMA,
            pltpu.VMEM_SHARED((NP, D_HID), jnp.float32),
            pltpu.VMEM_SHARED((NP, D_HID), jnp.float32),
        ],
        compiler_params=_sc_params(),
    )(p1, h_np, dinv_full, b1, norm, row_p, col_p, zeros16)


def _tc1_body(x_ref, w1_ref, h_ref):
    h_ref[...] = jnp.dot(x_ref[...], w1_ref[...],
                         preferred_element_type=jnp.float32)


def _tc1(x, W1):
    grid = NP // R_TC
    return pl.pallas_call(
        _tc1_body,
        grid=(grid,),
        in_specs=[
            pl.BlockSpec((R_TC, D_IN), lambda i: (i, 0)),
            pl.BlockSpec((D_IN, D_HID), lambda i: (0, 0)),
        ],
        out_specs=pl.BlockSpec((R_TC, D_HID), lambda i: (i, 0)),
        out_shape=jax.ShapeDtypeStruct((NP, D_HID), jnp.float32),
    )(x, W1)


def _tc3_body(p1_ref, p2_ref, h_ref, dinv_ref, b1_ref, w2_ref, b2_ref,
              out_ref):
    dsq = dinv_ref[...] * dinv_ref[...]
    h1 = jnp.maximum(
        p1_ref[0] + p1_ref[1] + dsq * h_ref[...] + b1_ref[...], 0.0)
    agg = p2_ref[0] + p2_ref[1] + dsq * h1
    z = jnp.dot(agg, w2_ref[...], preferred_element_type=jnp.float32)
    z = z + b2_ref[...]
    m = jnp.max(z, axis=1, keepdims=True)
    zs = z - m
    lse = jnp.log(jnp.sum(jnp.exp(zs), axis=1, keepdims=True))
    out_ref[...] = zs - lse


def _tc3(p1, p2, h_np, dinv_col, b1_row, W2, b2_row):
    grid = (N + R_TC - 1) // R_TC
    return pl.pallas_call(
        _tc3_body,
        grid=(grid,),
        in_specs=[
            pl.BlockSpec((NC, R_TC, D_HID), lambda i: (0, i, 0)),
            pl.BlockSpec((NC, R_TC, D_HID), lambda i: (0, i, 0)),
            pl.BlockSpec((R_TC, D_HID), lambda i: (i, 0)),
            pl.BlockSpec((R_TC, 1), lambda i: (i, 0)),
            pl.BlockSpec((1, D_HID), lambda i: (0, 0)),
            pl.BlockSpec((D_HID, D_OUT), lambda i: (0, 0)),
            pl.BlockSpec((1, D_OUT), lambda i: (0, 0)),
        ],
        out_specs=pl.BlockSpec((R_TC, D_OUT), lambda i: (i, 0)),
        out_shape=jax.ShapeDtypeStruct((N, D_OUT), jnp.float32),
    )(p1, p2, h_np, dinv_col, b1_row, W2, b2_row)


def kernel(x, edge_index, edge_weight, W1, b1, W2, b2):
    row = edge_index[0].astype(jnp.int32)
    col = edge_index[1].astype(jnp.int32)
    ew = edge_weight.astype(jnp.float32)

    pad = EPAD - E
    row_p = jnp.concatenate([row, jnp.zeros((pad,), jnp.int32)])
    col_p = jnp.concatenate([col, jnp.zeros((pad,), jnp.int32)])
    ew_p = jnp.concatenate([ew, jnp.zeros((pad,), jnp.float32)])
    col_p = col_p.reshape(NW * NCH, CHUNK)
    ew_p = ew_p.reshape(NW * NCH, CHUNK)

    dinv_full = _dinv_kernel(col_p, ew_p)
    h = _tc1(x, W1)
    dinv_col = dinv_full[:N].reshape(N, 1)
    zeros16 = jnp.zeros((NP, D_HID), jnp.float32)
    p1, norm = _msgn(h, dinv_full[:N], row_p, col_p, ew_p, zeros16)
    p2 = _msg_layer2(p1, h, dinv_full, b1, norm, row_p, col_p, zeros16)
    return _tc3(p1.reshape(NC, NP, D_HID), p2.reshape(NC, NP, D_HID),
                h, dinv_col, b1.reshape(1, D_HID), W2,
                b2.reshape(1, D_OUT))
